# baseline (speedup 1.0000x reference)
import functools

import jax
import jax.numpy as jnp
from jax import lax
from jax.experimental import pallas as pl
from jax.experimental.pallas import tpu as pltpu
from jax.experimental.pallas import tpu_sc as plsc

N = 10000
E = 320000
D = 128

NC = 2
NS = 16
NW = NC * NS
CHUNK = 128
C = 80
E_PAD = NW * CHUNK * C
N_PAD = 10112
ROWS_PER_TILE = N_PAD // NS


def _sc_aggregate_kernel(x_hbm, src_hbm, dst_hbm, zeros_hbm, out_hbm,
                         src_v, dst_v, rows_v, acc_sh, sem):
    cid = lax.axis_index("c")
    sid = lax.axis_index("s")
    wid = cid * NS + sid

    @pl.when(sid == 0)
    def _():
        pltpu.sync_copy(zeros_hbm, acc_sh)

    pltpu.sync_copy(src_hbm.at[wid], src_v)
    pltpu.sync_copy(dst_hbm.at[wid], dst_v)
    plsc.subcore_barrier()

    def body(j, carry):
        pltpu.async_copy(x_hbm.at[src_v.at[j]], rows_v, sem).wait()
        pltpu.sync_copy(rows_v, acc_sh.at[dst_v.at[j]], add=True)
        return carry

    lax.fori_loop(0, C, body, 0)
    plsc.subcore_barrier()

    base = sid * ROWS_PER_TILE
    pltpu.sync_copy(acc_sh.at[pl.ds(base, ROWS_PER_TILE)],
                    out_hbm.at[cid, pl.ds(base, ROWS_PER_TILE)])


_sc_aggregate = functools.partial(
    pl.kernel,
    out_type=jax.ShapeDtypeStruct((NC, N_PAD, D), jnp.float32),
    mesh=plsc.VectorSubcoreMesh(core_axis_name="c", subcore_axis_name="s"),
    scratch_types=[
        pltpu.VMEM((C, CHUNK), jnp.int32),
        pltpu.VMEM((C, CHUNK), jnp.int32),
        pltpu.VMEM((CHUNK, D), jnp.float32),
        pltpu.VMEM_SHARED((N_PAD, D), jnp.float32),
        pltpu.SemaphoreType.DMA,
    ],
)(_sc_aggregate_kernel)


def _tc_layer_kernel(a_ref, x_ref, wrel_ref, wroot_ref, b_ref, o_ref):
    a = a_ref[0] + a_ref[1]
    o_ref[...] = (
        jnp.dot(a, wrel_ref[...], preferred_element_type=jnp.float32)
        + jnp.dot(x_ref[...], wroot_ref[...], preferred_element_type=jnp.float32)
        + b_ref[...]
    )


def _tc_layer(aggr, xin, W_rel, W_root, b, out_rows, br):
    return pl.pallas_call(
        _tc_layer_kernel,
        out_shape=jax.ShapeDtypeStruct((out_rows, D), jnp.float32),
        grid=(out_rows // br,),
        in_specs=[
            pl.BlockSpec((NC, br, D), lambda i: (0, i, 0)),
            pl.BlockSpec((br, D), lambda i: (i, 0)),
            pl.BlockSpec((D, D), lambda i: (0, 0)),
            pl.BlockSpec((D, D), lambda i: (0, 0)),
            pl.BlockSpec((1, D), lambda i: (0, 0)),
        ],
        out_specs=pl.BlockSpec((br, D), lambda i: (i, 0)),
    )(aggr, xin, W_rel, W_root, b.reshape(1, D))


def kernel(x, edge_index, W_rel1, W_root1, b1, W_rel2, W_root2, b2):
    src = edge_index[0].astype(jnp.int32)
    dst = edge_index[1].astype(jnp.int32)
    pad = E_PAD - E
    src = jnp.concatenate([src, jnp.full((pad,), N, jnp.int32)]).reshape(NW, C, CHUNK)
    dst = jnp.concatenate([dst, jnp.full((pad,), N, jnp.int32)]).reshape(NW, C, CHUNK)

    zeros = jnp.zeros((N_PAD, D), jnp.float32)
    x_pad = zeros.at[:N].set(x)

    aggr1 = _sc_aggregate(x_pad, src, dst, zeros)
    h = _tc_layer(aggr1, x, W_rel1, W_root1, b1, N, 400)
    h_pad = zeros.at[:N].set(h)
    aggr2 = _sc_aggregate(h_pad, src, dst, zeros)
    out = _tc_layer(aggr2, h, W_rel2, W_root2, b2, N, 400)
    return out


# speedup vs baseline: 2.6946x; 2.6946x over previous
import functools

import jax
import jax.numpy as jnp
from jax import lax
from jax.experimental import pallas as pl
from jax.experimental.pallas import tpu as pltpu
from jax.experimental.pallas import tpu_sc as plsc

N = 10000
E = 320000
D = 128

NC = 2
NS = 16
NW = NC * NS
CHUNK = 128
C = 79
E_PAD = NW * CHUNK * C
N_PAD = 10112
ROWS_PER_TILE = N_PAD // NS


def _sc_aggregate_kernel(x_hbm, src_hbm, dst_hbm, zeros_hbm, out_hbm,
                         src_v, dst_v, rows_v, acc_sh, sem):
    cid = lax.axis_index("c")
    sid = lax.axis_index("s")
    wid = cid * NS + sid

    @pl.when(sid == 0)
    def _():
        pltpu.sync_copy(zeros_hbm, acc_sh)

    pltpu.sync_copy(src_hbm.at[wid], src_v)
    pltpu.sync_copy(dst_hbm.at[wid], dst_v)
    plsc.subcore_barrier()

    def body(j, carry):
        pltpu.async_copy(x_hbm.at[src_v.at[j]], rows_v, sem).wait()
        pltpu.sync_copy(rows_v, acc_sh.at[dst_v.at[j]], add=True)
        return carry

    lax.fori_loop(0, C, body, 0)
    plsc.subcore_barrier()

    base = sid * ROWS_PER_TILE
    pltpu.sync_copy(acc_sh.at[pl.ds(base, ROWS_PER_TILE)],
                    out_hbm.at[cid, pl.ds(base, ROWS_PER_TILE)])


_sc_aggregate = functools.partial(
    pl.kernel,
    out_type=jax.ShapeDtypeStruct((NC, N_PAD, D), jnp.float32),
    mesh=plsc.VectorSubcoreMesh(core_axis_name="c", subcore_axis_name="s"),
    scratch_types=[
        pltpu.VMEM((C, CHUNK), jnp.int32),
        pltpu.VMEM((C, CHUNK), jnp.int32),
        pltpu.VMEM((CHUNK, D), jnp.float32),
        pltpu.VMEM_SHARED((N_PAD, D), jnp.float32),
        pltpu.SemaphoreType.DMA,
    ],
)(_sc_aggregate_kernel)


def _tc_layer_kernel(a_ref, x_ref, wrel_ref, wroot_ref, b_ref, o_ref):
    a = a_ref[0] + a_ref[1]
    o_ref[...] = (
        jnp.dot(a, wrel_ref[...], preferred_element_type=jnp.float32)
        + jnp.dot(x_ref[...], wroot_ref[...], preferred_element_type=jnp.float32)
        + b_ref[...]
    )


def _tc_layer(aggr, xin, W_rel, W_root, b, out_rows, br):
    return pl.pallas_call(
        _tc_layer_kernel,
        out_shape=jax.ShapeDtypeStruct((out_rows, D), jnp.float32),
        grid=(out_rows // br,),
        in_specs=[
            pl.BlockSpec((NC, br, D), lambda i: (0, i, 0)),
            pl.BlockSpec((br, D), lambda i: (i, 0)),
            pl.BlockSpec((D, D), lambda i: (0, 0)),
            pl.BlockSpec((D, D), lambda i: (0, 0)),
            pl.BlockSpec((1, D), lambda i: (0, 0)),
        ],
        out_specs=pl.BlockSpec((br, D), lambda i: (i, 0)),
    )(aggr, xin, W_rel, W_root, b.reshape(1, D))


def kernel(x, edge_index, W_rel1, W_root1, b1, W_rel2, W_root2, b2):
    src = edge_index[0].astype(jnp.int32)
    dst = edge_index[1].astype(jnp.int32)
    pad = E_PAD - E
    dummy = N + (jnp.arange(pad, dtype=jnp.int32) % (N_PAD - N))
    src = jnp.concatenate([src, dummy]).reshape(NW, C, CHUNK)
    dst = jnp.concatenate([dst, dummy]).reshape(NW, C, CHUNK)

    zeros = jnp.zeros((N_PAD, D), jnp.float32)
    x_pad = zeros.at[:N].set(x)

    aggr1 = _sc_aggregate(x_pad, src, dst, zeros)
    h = _tc_layer(aggr1, x, W_rel1, W_root1, b1, N, 400)
    h_pad = zeros.at[:N].set(h)
    aggr2 = _sc_aggregate(h_pad, src, dst, zeros)
    out = _tc_layer(aggr2, h, W_rel2, W_root2, b2, N, 400)
    return out


# speedup vs baseline: 2.7655x; 1.0263x over previous
import functools

import jax
import jax.numpy as jnp
from jax import lax
from jax.experimental import pallas as pl
from jax.experimental.pallas import tpu as pltpu
from jax.experimental.pallas import tpu_sc as plsc

N = 10000
E = 320000
D = 128

NC = 2
NS = 16
NW = NC * NS
CHUNK = 128
C = 79
E_PAD = NW * CHUNK * C
N_PAD = 10112
ROWS_PER_TILE = N_PAD // NS


def _sc_aggregate_kernel(x_hbm, src_hbm, dst_hbm, zeros_hbm, out_hbm,
                         src_v, dst_v, rows_v, acc_sh, sem):
    cid = lax.axis_index("c")
    sid = lax.axis_index("s")
    wid = cid * NS + sid

    @pl.when(sid == 0)
    def _():
        pltpu.sync_copy(zeros_hbm, acc_sh)

    pltpu.sync_copy(src_hbm.at[wid], src_v)
    pltpu.sync_copy(dst_hbm.at[wid], dst_v)
    plsc.subcore_barrier()

    def body(j, carry):
        pltpu.async_copy(x_hbm.at[src_v.at[j]], rows_v, sem).wait()
        pltpu.sync_copy(rows_v, acc_sh.at[dst_v.at[j]], add=True)
        return carry

    lax.fori_loop(0, C, body, 0)
    plsc.subcore_barrier()

    base = sid * ROWS_PER_TILE
    pltpu.sync_copy(acc_sh.at[pl.ds(base, ROWS_PER_TILE)],
                    out_hbm.at[cid, pl.ds(base, ROWS_PER_TILE)])


_sc_aggregate = functools.partial(
    pl.kernel,
    out_type=jax.ShapeDtypeStruct((NC, N_PAD, D), jnp.float32),
    mesh=plsc.VectorSubcoreMesh(core_axis_name="c", subcore_axis_name="s"),
    scratch_types=[
        pltpu.VMEM((C, CHUNK), jnp.int32),
        pltpu.VMEM((C, CHUNK), jnp.int32),
        pltpu.VMEM((CHUNK, D), jnp.float32),
        pltpu.VMEM_SHARED((N_PAD, D), jnp.float32),
        pltpu.SemaphoreType.DMA,
    ],
)(_sc_aggregate_kernel)


def _tc_layer_kernel(a_ref, x_ref, wrel_ref, wroot_ref, b_ref, o_ref):
    a = a_ref[0] + a_ref[1]
    o_ref[...] = (
        jnp.dot(a, wrel_ref[...], preferred_element_type=jnp.float32)
        + jnp.dot(x_ref[...], wroot_ref[...], preferred_element_type=jnp.float32)
        + b_ref[...]
    )


def _tc_layer(aggr, xin, W_rel, W_root, b, out_rows, br):
    return pl.pallas_call(
        _tc_layer_kernel,
        out_shape=jax.ShapeDtypeStruct((out_rows, D), jnp.float32),
        grid=(out_rows // br,),
        in_specs=[
            pl.BlockSpec((NC, br, D), lambda i: (0, i, 0)),
            pl.BlockSpec((br, D), lambda i: (i, 0)),
            pl.BlockSpec((D, D), lambda i: (0, 0)),
            pl.BlockSpec((D, D), lambda i: (0, 0)),
            pl.BlockSpec((1, D), lambda i: (0, 0)),
        ],
        out_specs=pl.BlockSpec((br, D), lambda i: (i, 0)),
    )(aggr, xin, W_rel, W_root, b.reshape(1, D))


def kernel(x, edge_index, W_rel1, W_root1, b1, W_rel2, W_root2, b2):
    src = edge_index[0].astype(jnp.int32)
    dst = edge_index[1].astype(jnp.int32)
    pad = E_PAD - E
    dummy = N + (jnp.arange(pad, dtype=jnp.int32) % (N_PAD - N))
    src = jnp.concatenate([src, dummy]).reshape(NW, C, CHUNK)
    dst = jnp.concatenate([dst, dummy]).reshape(NW, C, CHUNK)

    zeros = jnp.zeros((N_PAD, D), jnp.float32)
    x_pad = zeros.at[:N].set(x)

    aggr1 = _sc_aggregate(x_pad, src, dst, zeros)
    h_pad = _tc_layer(aggr1, x_pad, W_rel1, W_root1, b1, N_PAD, N_PAD // 16)
    aggr2 = _sc_aggregate(h_pad, src, dst, zeros)
    out = _tc_layer(aggr2, h_pad, W_rel2, W_root2, b2, N, 400)
    return out


# speedup vs baseline: 3.7224x; 1.3460x over previous
import functools

import jax
import jax.numpy as jnp
from jax import lax
from jax.experimental import pallas as pl
from jax.experimental.pallas import tpu as pltpu
from jax.experimental.pallas import tpu_sc as plsc

N = 10000
E = 320000
D = 128

NC = 2
NS = 16
NW = NC * NS
CHUNK = 128
C = 80
HC = C // 2
K = 8
E_PAD = NW * CHUNK * C
N_PAD = 10112
ROWS_PER_TILE = N_PAD // NS


def _sc_aggregate_kernel(x_hbm, src_hbm, dst_hbm, zeros_hbm, out_hbm,
                         src_v, dst_v, rows_v, acc_sh, sem0, sem1):
    cid = lax.axis_index("c")
    sid = lax.axis_index("s")
    wid = cid * NS + sid

    @pl.when(sid == 0)
    def _():
        pltpu.sync_copy(zeros_hbm, acc_sh)

    for h in range(2):
        pltpu.sync_copy(src_hbm.at[wid, pl.ds(h * HC, HC)], src_v)
        pltpu.sync_copy(dst_hbm.at[wid, pl.ds(h * HC, HC)], dst_v)
        if h == 0:
            plsc.subcore_barrier()

        sems = (sem0, sem1)

        def group_body(g, carry):
            base = g * K
            cps = [None, None]
            cps[0] = pltpu.async_copy(x_hbm.at[src_v.at[base]], rows_v.at[0], sem0)
            for k in range(K):
                p = k % 2
                if k + 1 < K:
                    cps[1 - p] = pltpu.async_copy(
                        x_hbm.at[src_v.at[base + k + 1]], rows_v.at[1 - p],
                        sems[1 - p])
                cps[p].wait()
                pltpu.sync_copy(rows_v.at[p], acc_sh.at[dst_v.at[base + k]],
                                add=True)
            return carry

        lax.fori_loop(0, HC // K, group_body, 0)
    plsc.subcore_barrier()

    base = sid * ROWS_PER_TILE
    pltpu.sync_copy(acc_sh.at[pl.ds(base, ROWS_PER_TILE)],
                    out_hbm.at[cid, pl.ds(base, ROWS_PER_TILE)])


_sc_aggregate = functools.partial(
    pl.kernel,
    out_type=jax.ShapeDtypeStruct((NC, N_PAD, D), jnp.float32),
    mesh=plsc.VectorSubcoreMesh(core_axis_name="c", subcore_axis_name="s"),
    scratch_types=[
        pltpu.VMEM((HC, CHUNK), jnp.int32),
        pltpu.VMEM((HC, CHUNK), jnp.int32),
        pltpu.VMEM((2, CHUNK, D), jnp.float32),
        pltpu.VMEM_SHARED((N_PAD, D), jnp.float32),
        pltpu.SemaphoreType.DMA,
        pltpu.SemaphoreType.DMA,
    ],
)(_sc_aggregate_kernel)


def _tc_layer_kernel(a_ref, x_ref, wrel_ref, wroot_ref, b_ref, o_ref):
    a = a_ref[0] + a_ref[1]
    o_ref[...] = (
        jnp.dot(a, wrel_ref[...], preferred_element_type=jnp.float32)
        + jnp.dot(x_ref[...], wroot_ref[...], preferred_element_type=jnp.float32)
        + b_ref[...]
    )


def _tc_layer(aggr, xin, W_rel, W_root, b, out_rows, br):
    return pl.pallas_call(
        _tc_layer_kernel,
        out_shape=jax.ShapeDtypeStruct((out_rows, D), jnp.float32),
        grid=(out_rows // br,),
        in_specs=[
            pl.BlockSpec((NC, br, D), lambda i: (0, i, 0)),
            pl.BlockSpec((br, D), lambda i: (i, 0)),
            pl.BlockSpec((D, D), lambda i: (0, 0)),
            pl.BlockSpec((D, D), lambda i: (0, 0)),
            pl.BlockSpec((1, D), lambda i: (0, 0)),
        ],
        out_specs=pl.BlockSpec((br, D), lambda i: (i, 0)),
    )(aggr, xin, W_rel, W_root, b.reshape(1, D))


def kernel(x, edge_index, W_rel1, W_root1, b1, W_rel2, W_root2, b2):
    src = edge_index[0].astype(jnp.int32)
    dst = edge_index[1].astype(jnp.int32)
    pad = E_PAD - E
    dummy = N + (jnp.arange(pad, dtype=jnp.int32) % (N_PAD - N))
    src = jnp.concatenate([src, dummy]).reshape(NW, C, CHUNK)
    dst = jnp.concatenate([dst, dummy]).reshape(NW, C, CHUNK)

    zeros = jnp.zeros((N_PAD, D), jnp.float32)
    x_pad = zeros.at[:N].set(x)

    aggr1 = _sc_aggregate(x_pad, src, dst, zeros)
    h_pad = _tc_layer(aggr1, x_pad, W_rel1, W_root1, b1, N_PAD, N_PAD // 16)
    aggr2 = _sc_aggregate(h_pad, src, dst, zeros)
    out = _tc_layer(aggr2, h_pad, W_rel2, W_root2, b2, N, 400)
    return out


# speedup vs baseline: 3.9265x; 1.0548x over previous
import functools

import jax
import jax.numpy as jnp
from jax import lax
from jax.experimental import pallas as pl
from jax.experimental.pallas import tpu as pltpu
from jax.experimental.pallas import tpu_sc as plsc

N = 10000
E = 320000
D = 128

NC = 2
NS = 16
NW = NC * NS
CHUNK = 128
C = 80
HC = C // 2
K = 20
E_PAD = NW * CHUNK * C
N_PAD = 10112
ROWS_PER_TILE = N_PAD // NS


def _sc_aggregate_kernel(x_hbm, src_hbm, dst_hbm, zeros_hbm, out_hbm,
                         src_v, dst_v, rows_v, acc_sh, sem0, sem1):
    cid = lax.axis_index("c")
    sid = lax.axis_index("s")
    wid = cid * NS + sid

    @pl.when(sid == 0)
    def _():
        pltpu.sync_copy(zeros_hbm, acc_sh)

    for h in range(2):
        pltpu.sync_copy(src_hbm.at[wid, pl.ds(h * HC, HC)], src_v)
        pltpu.sync_copy(dst_hbm.at[wid, pl.ds(h * HC, HC)], dst_v)
        if h == 0:
            plsc.subcore_barrier()

        sems = (sem0, sem1)

        def group_body(g, carry):
            base = g * K
            cps = [None, None]
            cps[0] = pltpu.async_copy(x_hbm.at[src_v.at[base]], rows_v.at[0], sem0)
            for k in range(K):
                p = k % 2
                if k + 1 < K:
                    cps[1 - p] = pltpu.async_copy(
                        x_hbm.at[src_v.at[base + k + 1]], rows_v.at[1 - p],
                        sems[1 - p])
                cps[p].wait()
                pltpu.sync_copy(rows_v.at[p], acc_sh.at[dst_v.at[base + k]],
                                add=True)
            return carry

        lax.fori_loop(0, HC // K, group_body, 0)
    plsc.subcore_barrier()

    base = sid * ROWS_PER_TILE
    pltpu.sync_copy(acc_sh.at[pl.ds(base, ROWS_PER_TILE)],
                    out_hbm.at[cid, pl.ds(base, ROWS_PER_TILE)])


_sc_aggregate = functools.partial(
    pl.kernel,
    out_type=jax.ShapeDtypeStruct((NC, N_PAD, D), jnp.float32),
    mesh=plsc.VectorSubcoreMesh(core_axis_name="c", subcore_axis_name="s"),
    scratch_types=[
        pltpu.VMEM((HC, CHUNK), jnp.int32),
        pltpu.VMEM((HC, CHUNK), jnp.int32),
        pltpu.VMEM((2, CHUNK, D), jnp.float32),
        pltpu.VMEM_SHARED((N_PAD, D), jnp.float32),
        pltpu.SemaphoreType.DMA,
        pltpu.SemaphoreType.DMA,
    ],
)(_sc_aggregate_kernel)


def _tc_layer_kernel(a_ref, x_ref, wrel_ref, wroot_ref, b_ref, o_ref):
    a = a_ref[0] + a_ref[1]
    o_ref[...] = (
        jnp.dot(a, wrel_ref[...], preferred_element_type=jnp.float32)
        + jnp.dot(x_ref[...], wroot_ref[...], preferred_element_type=jnp.float32)
        + b_ref[...]
    )


def _tc_layer(aggr, xin, W_rel, W_root, b, out_rows, br):
    return pl.pallas_call(
        _tc_layer_kernel,
        out_shape=jax.ShapeDtypeStruct((out_rows, D), jnp.float32),
        grid=(out_rows // br,),
        in_specs=[
            pl.BlockSpec((NC, br, D), lambda i: (0, i, 0)),
            pl.BlockSpec((br, D), lambda i: (i, 0)),
            pl.BlockSpec((D, D), lambda i: (0, 0)),
            pl.BlockSpec((D, D), lambda i: (0, 0)),
            pl.BlockSpec((1, D), lambda i: (0, 0)),
        ],
        out_specs=pl.BlockSpec((br, D), lambda i: (i, 0)),
    )(aggr, xin, W_rel, W_root, b.reshape(1, D))


def kernel(x, edge_index, W_rel1, W_root1, b1, W_rel2, W_root2, b2):
    src = edge_index[0].astype(jnp.int32)
    dst = edge_index[1].astype(jnp.int32)
    pad = E_PAD - E
    dummy = N + (jnp.arange(pad, dtype=jnp.int32) % (N_PAD - N))
    src = jnp.concatenate([src, dummy]).reshape(NW, C, CHUNK)
    dst = jnp.concatenate([dst, dummy]).reshape(NW, C, CHUNK)

    zeros = jnp.zeros((N_PAD, D), jnp.float32)
    x_pad = zeros.at[:N].set(x)

    aggr1 = _sc_aggregate(x_pad, src, dst, zeros)
    h_pad = _tc_layer(aggr1, x_pad, W_rel1, W_root1, b1, N_PAD, N_PAD // 16)
    aggr2 = _sc_aggregate(h_pad, src, dst, zeros)
    out = _tc_layer(aggr2, h_pad, W_rel2, W_root2, b2, N, 400)
    return out


# speedup vs baseline: 3.9898x; 1.0161x over previous
import functools

import jax
import jax.numpy as jnp
from jax import lax
from jax.experimental import pallas as pl
from jax.experimental.pallas import tpu as pltpu
from jax.experimental.pallas import tpu_sc as plsc

N = 10000
E = 320000
D = 128

NC = 2
NS = 16
NW = NC * NS
CHUNK = 128
C = 80
HC = C // 2
K = 20
E_PAD = NW * CHUNK * C
N_PAD = 10112
ROWS_PER_TILE = N_PAD // NS


def _sc_aggregate_kernel(x_hbm, src_hbm, dst_hbm, zeros_hbm, out_hbm,
                         src_v, dst_v, rows_v, acc_sh, sem0, sem1):
    cid = lax.axis_index("c")
    sid = lax.axis_index("s")
    wid = cid * NS + sid

    zbase = sid * ROWS_PER_TILE
    pltpu.sync_copy(zeros_hbm.at[pl.ds(zbase, ROWS_PER_TILE)],
                    acc_sh.at[pl.ds(zbase, ROWS_PER_TILE)])

    for h in range(2):
        pltpu.sync_copy(src_hbm.at[wid, pl.ds(h * HC, HC)], src_v)
        pltpu.sync_copy(dst_hbm.at[wid, pl.ds(h * HC, HC)], dst_v)
        if h == 0:
            plsc.subcore_barrier()

        sems = (sem0, sem1)

        def group_body(g, carry):
            base = g * K
            cps = [None, None]
            cps[0] = pltpu.async_copy(x_hbm.at[src_v.at[base]], rows_v.at[0], sem0)
            for k in range(K):
                p = k % 2
                if k + 1 < K:
                    cps[1 - p] = pltpu.async_copy(
                        x_hbm.at[src_v.at[base + k + 1]], rows_v.at[1 - p],
                        sems[1 - p])
                cps[p].wait()
                pltpu.sync_copy(rows_v.at[p], acc_sh.at[dst_v.at[base + k]],
                                add=True)
            return carry

        lax.fori_loop(0, HC // K, group_body, 0)
    plsc.subcore_barrier()

    base = sid * ROWS_PER_TILE
    pltpu.sync_copy(acc_sh.at[pl.ds(base, ROWS_PER_TILE)],
                    out_hbm.at[cid, pl.ds(base, ROWS_PER_TILE)])


_sc_aggregate = functools.partial(
    pl.kernel,
    out_type=jax.ShapeDtypeStruct((NC, N_PAD, D), jnp.float32),
    mesh=plsc.VectorSubcoreMesh(core_axis_name="c", subcore_axis_name="s"),
    scratch_types=[
        pltpu.VMEM((HC, CHUNK), jnp.int32),
        pltpu.VMEM((HC, CHUNK), jnp.int32),
        pltpu.VMEM((2, CHUNK, D), jnp.float32),
        pltpu.VMEM_SHARED((N_PAD, D), jnp.float32),
        pltpu.SemaphoreType.DMA,
        pltpu.SemaphoreType.DMA,
    ],
)(_sc_aggregate_kernel)


def _tc_layer_kernel(a_ref, x_ref, wrel_ref, wroot_ref, b_ref, o_ref):
    a = a_ref[0] + a_ref[1]
    o_ref[...] = (
        jnp.dot(a, wrel_ref[...], preferred_element_type=jnp.float32)
        + jnp.dot(x_ref[...], wroot_ref[...], preferred_element_type=jnp.float32)
        + b_ref[...]
    )


def _tc_layer(aggr, xin, W_rel, W_root, b, out_rows, br):
    return pl.pallas_call(
        _tc_layer_kernel,
        out_shape=jax.ShapeDtypeStruct((out_rows, D), jnp.float32),
        grid=(out_rows // br,),
        in_specs=[
            pl.BlockSpec((NC, br, D), lambda i: (0, i, 0)),
            pl.BlockSpec((br, D), lambda i: (i, 0)),
            pl.BlockSpec((D, D), lambda i: (0, 0)),
            pl.BlockSpec((D, D), lambda i: (0, 0)),
            pl.BlockSpec((1, D), lambda i: (0, 0)),
        ],
        out_specs=pl.BlockSpec((br, D), lambda i: (i, 0)),
    )(aggr, xin, W_rel, W_root, b.reshape(1, D))


def kernel(x, edge_index, W_rel1, W_root1, b1, W_rel2, W_root2, b2):
    src = edge_index[0].astype(jnp.int32)
    dst = edge_index[1].astype(jnp.int32)
    pad = E_PAD - E
    dummy = N + (jnp.arange(pad, dtype=jnp.int32) % (N_PAD - N))
    src = jnp.concatenate([src, dummy]).reshape(NW, C, CHUNK)
    dst = jnp.concatenate([dst, dummy]).reshape(NW, C, CHUNK)

    zeros = jnp.zeros((N_PAD, D), jnp.float32)
    x_pad = zeros.at[:N].set(x)

    aggr1 = _sc_aggregate(x_pad, src, dst, zeros)
    h_pad = _tc_layer(aggr1, x_pad, W_rel1, W_root1, b1, N_PAD, N_PAD // 8)
    aggr2 = _sc_aggregate(h_pad, src, dst, zeros)
    out = _tc_layer(aggr2, h_pad, W_rel2, W_root2, b2, N, 400)
    return out
